# baseline (speedup 1.0000x reference)
import functools

import jax
import jax.numpy as jnp
from jax import lax
from jax.experimental import pallas as pl
from jax.experimental.pallas import tpu as pltpu
from jax.experimental.pallas import tpu_sc as plsc

B = 16384
D = 32
V = 1000000
NCOLS = 7812
TH = NCOLS * 128
ROWSTRIDE = NCOLS * 1024
FLAT = 4 * ROWSTRIDE


def _detile(ut_t, it_t):
    info = plsc.get_sparse_core_info()
    nc, ns = info.num_cores, info.num_subcores
    nw = nc * ns
    mesh = plsc.VectorSubcoreMesh(core_axis_name="c", subcore_axis_name="s")

    @functools.partial(
        pl.kernel,
        mesh=mesh,
        compiler_params=pltpu.CompilerParams(use_tc_tiling_on_sc=True),
        out_type=(
            jax.ShapeDtypeStruct((FLAT,), jnp.float32),
            jax.ShapeDtypeStruct((FLAT,), jnp.float32),
        ),
        scratch_types=[
            pltpu.SemaphoreType.DMA,
            pltpu.SemaphoreType.DMA,
        ],
    )
    def ka(ut_hbm, it_hbm, uflat_hbm, iflat_hbm, semu, semi):
        wid = lax.axis_index("s") * nc + lax.axis_index("c")
        nj_mine = (NCOLS - wid + nw - 1) // nw

        def jbody(t, _):
            j = wid + t * nw
            src_off = j * 128
            for d in range(D):
                dst = (d // 8) * ROWSTRIDE + (d % 8) * 128
                pltpu.async_copy(ut_hbm.at[d, pl.ds(src_off, 128)],
                                 uflat_hbm.at[pl.ds(j * 1024 + dst, 128)], semu)
                pltpu.async_copy(it_hbm.at[d, pl.ds(src_off, 128)],
                                 iflat_hbm.at[pl.ds(j * 1024 + dst, 128)], semi)
            return 0

        lax.fori_loop(0, nj_mine, jbody, 0)

        pltpu.make_async_copy(ut_hbm.at[0, pl.ds(0, 244 * 4096)],
                              uflat_hbm.at[pl.ds(0, 244 * 4096)], semu).wait()
        pltpu.make_async_copy(it_hbm.at[0, pl.ds(0, 244 * 4096)],
                              iflat_hbm.at[pl.ds(0, 244 * 4096)], semi).wait()

        @pl.when(wid < NCOLS - 244 * nw)
        def _():
            pltpu.make_async_copy(ut_hbm.at[0, pl.ds(0, 4096)],
                                  uflat_hbm.at[pl.ds(0, 4096)], semu).wait()
            pltpu.make_async_copy(it_hbm.at[0, pl.ds(0, 4096)],
                                  iflat_hbm.at[pl.ds(0, 4096)], semi).wait()

    return ka(ut_t, it_t)


def _gather_dot(uid, iid, uflat, iflat, tailu, taili):
    info = plsc.get_sparse_core_info()
    nc, ns = info.num_cores, info.num_subcores
    nw = nc * ns
    bpw = B // nw
    ng = bpw // 16
    mesh = plsc.VectorSubcoreMesh(core_axis_name="c", subcore_axis_name="s")

    @functools.partial(
        pl.kernel,
        mesh=mesh,
        compiler_params=pltpu.CompilerParams(
            use_tc_tiling_on_sc=False, needs_layout_passes=False),
        out_type=jax.ShapeDtypeStruct((B,), jnp.float32),
        scratch_types=[
            pltpu.VMEM((bpw,), jnp.int32),
            pltpu.VMEM((bpw,), jnp.int32),
            pltpu.VMEM((D * bpw,), jnp.int32),
            pltpu.VMEM((D * bpw,), jnp.int32),
            pltpu.VMEM((D, bpw), jnp.float32),
            pltpu.VMEM((D, bpw), jnp.float32),
            pltpu.VMEM((D, 128), jnp.float32),
            pltpu.VMEM((D, 128), jnp.float32),
            pltpu.VMEM((bpw,), jnp.float32),
            pltpu.SemaphoreType.DMA,
            pltpu.SemaphoreType.DMA,
        ],
    )
    def kb(uid_hbm, iid_hbm, uflat_hbm, iflat_hbm, tailu_hbm, taili_hbm, out_hbm,
           uid_v, iid_v, lu_v, li_v, du_v, di_v, tu_v, ti_v, out_v, semu, semi):
        wid = lax.axis_index("s") * nc + lax.axis_index("c")
        base = wid * bpw
        pltpu.sync_copy(uid_hbm.at[pl.ds(base, bpw)], uid_v)
        pltpu.sync_copy(iid_hbm.at[pl.ds(base, bpw)], iid_v)
        pltpu.sync_copy(tailu_hbm, tu_v)
        pltpu.sync_copy(taili_hbm, ti_v)

        def build(g, _):
            u = jnp.minimum(uid_v[pl.ds(g * 16, 16)], TH - 1)
            i = jnp.minimum(iid_v[pl.ds(g * 16, 16)], TH - 1)
            qu = ((u >> 7) << 10) + (u & 127)
            qi = ((i >> 7) << 10) + (i & 127)
            for d in range(D):
                c = (d // 8) * ROWSTRIDE + (d % 8) * 128
                lu_v[pl.ds(d * bpw + g * 16, 16)] = qu + c
                li_v[pl.ds(d * bpw + g * 16, 16)] = qi + c
            return 0

        lax.fori_loop(0, ng, build, 0)

        for d in range(D):
            pltpu.async_copy(uflat_hbm.at[lu_v.at[pl.ds(d * bpw, bpw)]], du_v.at[d], semu)
            pltpu.async_copy(iflat_hbm.at[li_v.at[pl.ds(d * bpw, bpw)]], di_v.at[d], semi)

        pltpu.make_async_copy(uflat_hbm.at[pl.ds(0, D * bpw)], du_v, semu).wait()
        pltpu.make_async_copy(iflat_hbm.at[pl.ds(0, D * bpw)], di_v, semi).wait()

        def dot(g, _):
            u16 = uid_v[pl.ds(g * 16, 16)]
            i16 = iid_v[pl.ds(g * 16, 16)]
            umask = u16 >= TH
            imask = i16 >= TH
            uti = jnp.maximum(u16 - TH, 0)
            iti = jnp.maximum(i16 - TH, 0)
            acc = jnp.zeros((16,), jnp.float32)
            for d in range(D):
                dvec = jnp.full((16,), d, jnp.int32)
                u = du_v[d, pl.ds(g * 16, 16)]
                i = di_v[d, pl.ds(g * 16, 16)]
                tu = plsc.load_gather(tu_v, [dvec, uti])
                ti = plsc.load_gather(ti_v, [dvec, iti])
                u = jnp.where(umask, tu, u)
                i = jnp.where(imask, ti, i)
                acc += u * i
            out_v[pl.ds(g * 16, 16)] = acc
            return 0

        lax.fori_loop(0, ng, dot, 0)
        pltpu.sync_copy(out_v, out_hbm.at[pl.ds(base, bpw)])

    return kb(uid, iid, uflat, iflat, tailu, taili)


def kernel(input_userID, input_itemID, user_table, item_table):
    uid = input_userID.astype(jnp.int32)
    iid = input_itemID.astype(jnp.int32)
    tailu = jnp.pad(user_table[TH:].T, ((0, 0), (0, 128 - (V - TH))))
    taili = jnp.pad(item_table[TH:].T, ((0, 0), (0, 128 - (V - TH))))
    uflat, iflat = _detile(user_table.T, item_table.T)
    return _gather_dot(uid, iid, uflat, iflat, tailu, taili)


# speedup vs baseline: 8.5865x; 8.5865x over previous
import functools

import jax
import jax.numpy as jnp
from jax import lax
from jax.experimental import pallas as pl
from jax.experimental.pallas import tpu as pltpu
from jax.experimental.pallas import tpu_sc as plsc

B = 16384
D = 32
V = 1000000


def _sc_fused(uid, iid, ut, it):
    info = plsc.get_sparse_core_info()
    nc, ns = info.num_cores, info.num_subcores
    nw = nc * ns
    bpw = B // nw
    ng = bpw // 16
    mesh = plsc.VectorSubcoreMesh(core_axis_name="c", subcore_axis_name="s")

    @functools.partial(
        pl.kernel,
        mesh=mesh,
        compiler_params=pltpu.CompilerParams(
            use_tc_tiling_on_sc=False, needs_layout_passes=False),
        out_type=jax.ShapeDtypeStruct((B,), jnp.float32),
        scratch_types=[
            pltpu.VMEM((bpw,), jnp.int32),
            pltpu.VMEM((bpw,), jnp.int32),
            pltpu.VMEM((bpw, D), jnp.float32),
            pltpu.VMEM((bpw, D), jnp.float32),
            pltpu.VMEM((bpw,), jnp.float32),
            pltpu.SemaphoreType.DMA,
            pltpu.SemaphoreType.DMA,
        ],
    )
    def k(uid_hbm, iid_hbm, ut_hbm, it_hbm, out_hbm,
          uid_v, iid_v, du_v, di_v, out_v, semu, semi):
        wid = lax.axis_index("s") * nc + lax.axis_index("c")
        base = wid * bpw
        pltpu.sync_copy(uid_hbm.at[pl.ds(base, bpw)], uid_v)
        pltpu.sync_copy(iid_hbm.at[pl.ds(base, bpw)], iid_v)

        cu = pltpu.async_copy(ut_hbm.at[uid_v], du_v, semu)
        ci = pltpu.async_copy(it_hbm.at[iid_v], di_v, semi)
        cu.wait()
        ci.wait()

        def dot(g, _):
            rows = g * 16 + lax.iota(jnp.int32, 16)
            acc = jnp.zeros((16,), jnp.float32)
            for d in range(D):
                dvec = jnp.full((16,), d, jnp.int32)
                u = plsc.load_gather(du_v, [rows, dvec])
                i = plsc.load_gather(di_v, [rows, dvec])
                acc += u * i
            out_v[pl.ds(g * 16, 16)] = acc
            return 0

        lax.fori_loop(0, ng, dot, 0)
        pltpu.sync_copy(out_v, out_hbm.at[pl.ds(base, bpw)])

    return k(uid, iid, ut, it)


def kernel(input_userID, input_itemID, user_table, item_table):
    uid = input_userID.astype(jnp.int32)
    iid = input_itemID.astype(jnp.int32)
    return _sc_fused(uid, iid, user_table, item_table)
